# baseline (speedup 1.0000x reference)
import jax
import jax.numpy as jnp
from jax import lax
from jax.experimental import pallas as pl
from jax.experimental.pallas import tpu as pltpu
from jax.experimental.pallas import tpu_sc as plsc

B = 16384
D = 128
NC = 2
NS = 16
NW = NC * NS
BPW = B // NW
LANES = 16
NCH = 8
C = BPW // NCH
NBUF = 2


def _double(buf):
    def row_fn(r, carry):
        for j in range(D // LANES):
            sl = (r, pl.ds(j * LANES, LANES))
            v = buf[sl]
            buf[sl] = v + v
        return carry

    lax.fori_loop(0, C, row_fn, 0, unroll=2)


def _body(x_hbm, idx_hbm, out_hbm, idx_v, buf0, buf1, gsem, ssem):
    wid = lax.axis_index("s") * NC + lax.axis_index("c")
    base = wid * BPW
    bufs = (buf0, buf1)

    pltpu.sync_copy(idx_hbm.at[wid], idx_v)

    gathers = [None] * NCH
    scatters = [None] * NCH
    gathers[0] = pltpu.async_copy(x_hbm.at[idx_v.at[0]], bufs[0], gsem)
    for c in range(NCH):
        buf = bufs[c % NBUF]
        if c >= NBUF - 1 and c + 1 < NCH:
            scatters[c + 1 - NBUF].wait()
        if c + 1 < NCH:
            gathers[c + 1] = pltpu.async_copy(
                x_hbm.at[idx_v.at[c + 1]], bufs[(c + 1) % NBUF], gsem
            )
        gathers[c].wait()
        _double(buf)
        scatters[c] = pltpu.async_copy(
            buf, out_hbm.at[pl.ds(base + c * C, C)], ssem
        )
    scatters[NCH - 2].wait()
    scatters[NCH - 1].wait()


def kernel(input, indices):
    idx32 = indices.astype(jnp.int32).reshape(NW, NCH, C)
    mesh = plsc.VectorSubcoreMesh(core_axis_name="c", subcore_axis_name="s")
    f = pl.kernel(
        _body,
        mesh=mesh,
        out_type=jax.ShapeDtypeStruct((B, D), jnp.float32),
        scratch_types=[
            pltpu.VMEM((NCH, C), jnp.int32),
            pltpu.VMEM((C, D), jnp.float32),
            pltpu.VMEM((C, D), jnp.float32),
            pltpu.SemaphoreType.DMA,
            pltpu.SemaphoreType.DMA,
        ],
    )
    return f(input, idx32)


# speedup vs baseline: 1.1228x; 1.1228x over previous
import jax
import jax.numpy as jnp
from jax import lax
from jax.experimental import pallas as pl
from jax.experimental.pallas import tpu as pltpu
from jax.experimental.pallas import tpu_sc as plsc

B = 16384
D = 128
NC = 2
NS = 16
NW = NC * NS
BPW = B // NW
LANES = 16
NCH = 2
C = BPW // NCH


def _double(buf):
    def row_fn(r, carry):
        for j in range(D // LANES):
            sl = (r, pl.ds(j * LANES, LANES))
            v = buf[sl]
            buf[sl] = v + v
        return carry

    lax.fori_loop(0, C, row_fn, 0, unroll=2)


def _body(x_hbm, idx_hbm, out_hbm, idx_v, buf0, buf1, gsem, ssem):
    wid = lax.axis_index("s") * NC + lax.axis_index("c")
    base = wid * BPW
    bufs = (buf0, buf1)

    pltpu.sync_copy(idx_hbm.at[pl.ds(base, BPW)], idx_v)

    g0 = pltpu.async_copy(x_hbm.at[idx_v.at[pl.ds(0, C)]], bufs[0], gsem)
    g1 = pltpu.async_copy(x_hbm.at[idx_v.at[pl.ds(C, C)]], bufs[1], gsem)
    g0.wait()
    _double(bufs[0])
    s0 = pltpu.async_copy(bufs[0], out_hbm.at[pl.ds(base, C)], ssem)
    g1.wait()
    _double(bufs[1])
    s1 = pltpu.async_copy(bufs[1], out_hbm.at[pl.ds(base + C, C)], ssem)
    s0.wait()
    s1.wait()


def kernel(input, indices):
    idx32 = indices.astype(jnp.int32)
    mesh = plsc.VectorSubcoreMesh(core_axis_name="c", subcore_axis_name="s")
    f = pl.kernel(
        _body,
        mesh=mesh,
        out_type=jax.ShapeDtypeStruct((B, D), jnp.float32),
        scratch_types=[
            pltpu.VMEM((BPW,), jnp.int32),
            pltpu.VMEM((C, D), jnp.float32),
            pltpu.VMEM((C, D), jnp.float32),
            pltpu.SemaphoreType.DMA,
            pltpu.SemaphoreType.DMA,
        ],
    )
    return f(input, idx32)
